# baseline (speedup 1.0000x reference)
import functools

import jax
import jax.numpy as jnp
from jax import lax
from jax.experimental import pallas as pl
from jax.experimental.pallas import tpu as pltpu
from jax.experimental.pallas import tpu_sc as plsc

_B = 4096
_L = 200
_D = 128
_BT = _B * _L

_NC = 2
_NS = 16
_NW = _NC * _NS
_CH = 128
_RPW = _BT // _NW
_CPW = _RPW // _CH
_NBUF = 4
assert _RPW % _CH == 0 and _CPW % _NBUF == 0

_mesh = plsc.VectorSubcoreMesh(core_axis_name="c", subcore_axis_name="s")


@functools.partial(
    pl.kernel,
    mesh=_mesh,
    out_type=jax.ShapeDtypeStruct((_BT, _D), jnp.float32),
    scratch_types=[
        pltpu.VMEM((_CPW, _CH), jnp.int32),
        *([pltpu.VMEM((_CH, _D), jnp.float32)] * _NBUF),
        pltpu.VMEM_SHARED((_NS, 2, _CH, _D), jnp.float32),
        *([pltpu.SemaphoreType.DMA] * (3 * _NBUF)),
    ],
)
def _gather_k(idx_hbm, table_hbm, out_hbm, idx_v, *bufs_and_sems):
    rows = bufs_and_sems[:_NBUF]
    spmem = bufs_and_sems[_NBUF]
    sems = bufs_and_sems[_NBUF + 1:]
    gsem = sems[:_NBUF]
    xsem = sems[_NBUF:2 * _NBUF]
    ssem = sems[2 * _NBUF:]

    wid = lax.axis_index("s") * _NC + lax.axis_index("c")
    pltpu.sync_copy(idx_hbm.at[pl.ds(wid * _CPW, _CPW)], idx_v)

    def out_slice(c):
        return out_hbm.at[pl.ds((wid * _CPW + c) * _CH, _CH)]

    def start_gather(c, b):
        pltpu.async_copy(table_hbm.at[idx_v.at[c]], rows[b], gsem[b])

    def wait_gather(c, b):
        pltpu.make_async_copy(table_hbm.at[idx_v.at[c]], rows[b], gsem[b]).wait()

    sid = lax.axis_index("s")

    def start_xbar(b, s):
        pltpu.async_copy(rows[b], spmem.at[sid, s], xsem[s])

    def wait_xbar(b, s):
        pltpu.make_async_copy(rows[b], spmem.at[sid, s], xsem[s]).wait()

    def start_store(c, s):
        pltpu.async_copy(spmem.at[sid, s], out_slice(c), ssem[s])

    def wait_store(c, s):
        pltpu.make_async_copy(spmem.at[sid, s], out_slice(c), ssem[s]).wait()

    def body(it, carry):
        c0 = it * _NBUF
        for b in range(_NBUF):
            start_gather(c0 + b, b)
        for b in range(_NBUF):
            s = b % 2
            wait_gather(c0 + b, b)
            if b < 2:

                @pl.when(it > 0)
                def _(s=s, c0=c0):
                    wait_store(c0, s)

            else:
                wait_store(c0 + b - 2, s)
            start_xbar(b, s)
            wait_xbar(b, s)
            start_store(c0 + b, s)
        return carry

    lax.fori_loop(0, _CPW // _NBUF, body, 0)
    for s in range(2):
        wait_store(_CPW - 2 + s, s)


def kernel(inp, table):
    idx = inp[..., 0].astype(jnp.int32).reshape(_NW * _CPW, _CH)
    out = _gather_k(idx, table)
    return out.reshape(_B, _L, _D)


# speedup vs baseline: 1.0653x; 1.0653x over previous
import functools

import jax
import jax.numpy as jnp
from jax import lax
from jax.experimental import pallas as pl
from jax.experimental.pallas import tpu as pltpu
from jax.experimental.pallas import tpu_sc as plsc

_B = 4096
_L = 200
_D = 128
_BT = _B * _L

_NC = 2
_NS = 16
_NW = _NC * _NS
_CH = 256
_CR = 1
_RPW = _BT // _NW
_CPW = _RPW // _CH
_PPW = _CPW // _CR
_NBUF = 2
assert _RPW % (_CR * _CH) == 0 and _PPW % _NBUF == 0

_mesh = plsc.VectorSubcoreMesh(core_axis_name="c", subcore_axis_name="s")


@functools.partial(
    pl.kernel,
    mesh=_mesh,
    out_type=jax.ShapeDtypeStruct((_BT, _D), jnp.float32),
    scratch_types=[
        pltpu.VMEM((_RPW,), jnp.int32),
        *([pltpu.VMEM((_CH, _D), jnp.float32)] * _NBUF),
        *([pltpu.SemaphoreType.DMA] * (2 * _NBUF)),
    ],
)
def _gather_k(idx_hbm, table_hbm, out_hbm, idx_v, *bufs_and_sems):
    rows = bufs_and_sems[:_NBUF]
    gsem = bufs_and_sems[_NBUF:2 * _NBUF]
    ssem = bufs_and_sems[2 * _NBUF:]

    wid = lax.axis_index("s") * _NC + lax.axis_index("c")
    pltpu.sync_copy(idx_hbm.at[pl.ds(wid * _RPW, _RPW)], idx_v)

    def out_slice(q):
        return out_hbm.at[pl.ds((wid * _PPW + q) * _CH, _CH)]

    def idx_slice(q):
        return idx_v.at[pl.ds(q * _CH, _CH)]

    def start_gather(q, b):
        pltpu.async_copy(table_hbm.at[idx_slice(q)], rows[b], gsem[b])

    def wait_gather(q, b):
        pltpu.make_async_copy(table_hbm.at[idx_slice(q)], rows[b], gsem[b]).wait()

    def start_store(q, b):
        pltpu.async_copy(rows[b], out_slice(q), ssem[b])

    def wait_store(q, b):
        pltpu.make_async_copy(rows[b], out_slice(q), ssem[b]).wait()

    def body(it, carry):
        q0 = it * _NBUF
        for b in range(_NBUF):

            @pl.when(it > 0)
            def _(b=b, q0=q0):
                wait_store(q0, b)

            start_gather(q0 + b, b)
        for b in range(_NBUF):
            wait_gather(q0 + b, b)
            start_store(q0 + b, b)
        return carry

    lax.fori_loop(0, _PPW // _NBUF, body, 0)
    for b in range(_NBUF):
        wait_store(_PPW - _NBUF + b, b)


def kernel(inp, table):
    idx = inp[..., 0].astype(jnp.int32).reshape(_BT)
    out = _gather_k(idx, table)
    return out.reshape(_B, _L, _D)


# speedup vs baseline: 1.0807x; 1.0144x over previous
import functools

import jax
import jax.numpy as jnp
from jax import lax
from jax.experimental import pallas as pl
from jax.experimental.pallas import tpu as pltpu
from jax.experimental.pallas import tpu_sc as plsc

_B = 4096
_L = 200
_D = 128
_BT = _B * _L

_NC = 2
_NS = 16
_NW = _NC * _NS
_CH = 128
_RPW = _BT // _NW
_CPW = _RPW // _CH
_NBUF = 4
assert _RPW % _CH == 0 and _CPW % _NBUF == 0 and _NBUF % 2 == 0

_mesh = plsc.VectorSubcoreMesh(core_axis_name="c", subcore_axis_name="s")


@functools.partial(
    pl.kernel,
    mesh=_mesh,
    out_type=jax.ShapeDtypeStruct((_BT, _D), jnp.float32),
    scratch_types=[
        pltpu.VMEM((_CPW, _CH), jnp.int32),
        *([pltpu.VMEM((_CH, _D), jnp.float32)] * _NBUF),
        *([pltpu.SemaphoreType.DMA] * (2 * _NBUF)),
    ],
)
def _gather_k(idx_hbm, table_hbm, out_hbm, idx_v, *bufs_and_sems):
    rows = bufs_and_sems[:_NBUF]
    gsem = bufs_and_sems[_NBUF:2 * _NBUF]
    ssem = bufs_and_sems[2 * _NBUF:]

    wid = lax.axis_index("s") * _NC + lax.axis_index("c")
    pltpu.sync_copy(idx_hbm.at[pl.ds(wid * _CPW, _CPW)], idx_v)

    def out_slice(c):
        return out_hbm.at[pl.ds((wid * _CPW + c) * _CH, _CH)]

    def start_gather(c, b):
        pltpu.async_copy(table_hbm.at[idx_v.at[c]], rows[b], gsem[b])

    def wait_gather(c, b):
        pltpu.make_async_copy(table_hbm.at[idx_v.at[c]], rows[b], gsem[b]).wait()

    def start_store(c, b):
        pltpu.async_copy(rows[b], out_slice(c), ssem[b])

    def wait_store(c, b):
        pltpu.make_async_copy(rows[b], out_slice(c), ssem[b]).wait()

    def body(it, carry):
        c0 = it * _NBUF
        for p in range(_NBUF // 2):
            b0, b1 = 2 * p, 2 * p + 1

            @pl.when(it > 0)
            def _(b0=b0, b1=b1, c0=c0):
                wait_store(c0, b0)
                wait_store(c0, b1)

            start_gather(c0 + b0, b0)
            start_gather(c0 + b1, b1)
            wait_gather(c0 + b0, b0)
            start_store(c0 + b0, b0)
            wait_gather(c0 + b1, b1)
            start_store(c0 + b1, b1)
        return carry

    lax.fori_loop(0, _CPW // _NBUF, body, 0)
    for b in range(_NBUF):
        wait_store(_CPW - _NBUF + b, b)


def kernel(inp, table):
    idx = inp[..., 0].astype(jnp.int32).reshape(_NW * _CPW, _CH)
    out = _gather_k(idx, table)
    return out.reshape(_B, _L, _D)


# speedup vs baseline: 1.0930x; 1.0114x over previous
import functools

import jax
import jax.numpy as jnp
from jax import lax
from jax.experimental import pallas as pl
from jax.experimental.pallas import tpu as pltpu
from jax.experimental.pallas import tpu_sc as plsc

_B = 4096
_L = 200
_D = 128
_BT = _B * _L

_NC = 2
_NS = 16
_NW = _NC * _NS
_CH = 128
_RPW = _BT // _NW
_CPW = _RPW // _CH
_NBUF = 4
assert _RPW % _CH == 0 and _CPW % _NBUF == 0 and _NBUF % 2 == 0

_mesh = plsc.VectorSubcoreMesh(core_axis_name="c", subcore_axis_name="s")


@functools.partial(
    pl.kernel,
    mesh=_mesh,
    out_type=jax.ShapeDtypeStruct((_BT, _D), jnp.float32),
    scratch_types=[
        pltpu.VMEM((_CPW, _CH), jnp.int32),
        *([pltpu.VMEM((_CH, _D), jnp.float32)] * _NBUF),
        *([pltpu.SemaphoreType.DMA] * (2 * _NBUF)),
    ],
)
def _gather_k(idx_hbm, table_hbm, out_hbm, idx_v, *bufs_and_sems):
    rows = bufs_and_sems[:_NBUF]
    gsem = bufs_and_sems[_NBUF:2 * _NBUF]
    ssem = bufs_and_sems[2 * _NBUF:]

    wid = lax.axis_index("s") * _NC + lax.axis_index("c")
    pltpu.sync_copy(idx_hbm.at[pl.ds(wid * _CPW, _CPW)], idx_v)

    def out_slice(c):
        return out_hbm.at[pl.ds((c * _NW + wid) * _CH, _CH)]

    def start_gather(c, b):
        pltpu.async_copy(table_hbm.at[idx_v.at[c]], rows[b], gsem[b])

    def wait_gather(c, b):
        pltpu.make_async_copy(table_hbm.at[idx_v.at[c]], rows[b], gsem[b]).wait()

    def start_store(c, b):
        pltpu.async_copy(rows[b], out_slice(c), ssem[b])

    def wait_store(c, b):
        pltpu.make_async_copy(rows[b], out_slice(c), ssem[b]).wait()

    def body(it, carry):
        c0 = it * _NBUF
        for p in range(_NBUF // 2):
            b0, b1 = 2 * p, 2 * p + 1

            @pl.when(it > 0)
            def _(b0=b0, b1=b1, c0=c0):
                wait_store(c0, b0)
                wait_store(c0, b1)

            start_gather(c0 + b0, b0)
            start_gather(c0 + b1, b1)
            wait_gather(c0 + b0, b0)
            start_store(c0 + b0, b0)
            wait_gather(c0 + b1, b1)
            start_store(c0 + b1, b1)
        return carry

    lax.fori_loop(0, _CPW // _NBUF, body, 0)
    for b in range(_NBUF):
        wait_store(_CPW - _NBUF + b, b)


def kernel(inp, table):
    idx = inp[..., 0].astype(jnp.int32).reshape(_CPW, _NW, _CH)
    idx = idx.transpose(1, 0, 2).reshape(_NW * _CPW, _CH)
    out = _gather_k(idx, table)
    return out.reshape(_B, _L, _D)
